# baseline (speedup 1.0000x reference)
import jax
import jax.numpy as jnp
from jax.experimental import pallas as pl
from jax.experimental.pallas import tpu as pltpu

_N = 4096
_TILE = 512
_HALF_STEPS = (_N // _TILE) // 2


def _fused_body(attr_ref, maska_ref, maskb_ref, w_ref, root_ref, bias_ref,
                outa_ref, outb_ref, p_ref):
    j = pl.program_id(0)

    @pl.when(j == 0)
    def _():
        p_ref[...] = jnp.dot(attr_ref[...], w_ref[...],
                             preferred_element_type=jnp.float32)

    attr_a = attr_ref[pl.ds(j * _TILE, _TILE), :]
    attr_b = attr_ref[pl.ds((j + _HALF_STEPS) * _TILE, _TILE), :]
    outa_ref[...] = (
        jnp.dot(maska_ref[...], p_ref[...], preferred_element_type=jnp.float32)
        + jnp.dot(attr_a, root_ref[...], preferred_element_type=jnp.float32)
        + bias_ref[...]
    )
    outb_ref[...] = (
        jnp.dot(maskb_ref[...], p_ref[...], preferred_element_type=jnp.float32)
        + jnp.dot(attr_b, root_ref[...], preferred_element_type=jnp.float32)
        + bias_ref[...]
    )


@jax.jit
def kernel(node_attr, node_mask, node_weight, root, bias):
    n, d_in = node_attr.shape
    d_out = node_weight.shape[1]
    bias2d = bias.reshape(1, d_out)
    half = n // 2

    grid = (_HALF_STEPS,)
    outa, outb = pl.pallas_call(
        _fused_body,
        grid=grid,
        in_specs=[
            pl.BlockSpec((n, d_in), lambda j: (0, 0)),
            pl.BlockSpec((_TILE, n), lambda j: (j, 0)),
            pl.BlockSpec((_TILE, n), lambda j: (j + _HALF_STEPS, 0)),
            pl.BlockSpec((d_in, d_out), lambda j: (0, 0)),
            pl.BlockSpec((d_in, d_out), lambda j: (0, 0)),
            pl.BlockSpec((1, d_out), lambda j: (0, 0)),
        ],
        out_specs=[
            pl.BlockSpec((_TILE, d_out), lambda j: (j, 0)),
            pl.BlockSpec((_TILE, d_out), lambda j: (j, 0)),
        ],
        out_shape=[
            jax.ShapeDtypeStruct((half, d_out), jnp.float32),
            jax.ShapeDtypeStruct((half, d_out), jnp.float32),
        ],
        scratch_shapes=[pltpu.VMEM((n, d_out), jnp.float32)],
    )(node_attr, node_mask, node_mask, node_weight, root, bias2d)
    return jnp.concatenate([outa, outb], axis=0)


# speedup vs baseline: 1.1600x; 1.1600x over previous
import jax
import jax.numpy as jnp
from jax.experimental import pallas as pl
from jax.experimental.pallas import tpu as pltpu

_N = 4096
_TILE = 512
_BUFS = 3


def _outer(attr_ref, mask_hbm, w_ref, root_ref, bias_ref, out_hbm,
           p_ref, r_ref):
    p_ref[...] = jnp.dot(attr_ref[...], w_ref[...],
                         preferred_element_type=jnp.float32)
    r_ref[...] = jnp.dot(attr_ref[...], root_ref[...],
                         preferred_element_type=jnp.float32) + bias_ref[...]

    def _inner(idxs, mask_blk, out_blk):
        i = idxs[0]
        out_blk[...] = (
            jnp.dot(mask_blk[...], p_ref[...],
                    preferred_element_type=jnp.float32)
            + r_ref[pl.ds(i * _TILE, _TILE), :]
        )

    pipe = pltpu.emit_pipeline(
        _inner,
        grid=(_N // _TILE,),
        in_specs=[pl.BlockSpec((_TILE, _N), lambda i: (i, 0),
                               pipeline_mode=pl.Buffered(buffer_count=_BUFS))],
        out_specs=[pl.BlockSpec((_TILE, 256), lambda i: (i, 0))],
        _explicit_indices=True,
    )
    pipe(mask_hbm, out_hbm)


@jax.jit
def kernel(node_attr, node_mask, node_weight, root, bias):
    n, d_in = node_attr.shape
    d_out = node_weight.shape[1]
    bias2d = bias.reshape(1, d_out)

    return pl.pallas_call(
        _outer,
        in_specs=[
            pl.BlockSpec(memory_space=pltpu.MemorySpace.VMEM),
            pl.BlockSpec(memory_space=pltpu.MemorySpace.HBM),
            pl.BlockSpec(memory_space=pltpu.MemorySpace.VMEM),
            pl.BlockSpec(memory_space=pltpu.MemorySpace.VMEM),
            pl.BlockSpec(memory_space=pltpu.MemorySpace.VMEM),
        ],
        out_specs=pl.BlockSpec(memory_space=pltpu.MemorySpace.HBM),
        out_shape=jax.ShapeDtypeStruct((n, d_out), jnp.float32),
        scratch_shapes=[
            pltpu.VMEM((n, d_out), jnp.float32),
            pltpu.VMEM((n, d_out), jnp.float32),
        ],
    )(node_attr, node_mask, node_weight, root, bias2d)


# speedup vs baseline: 1.2526x; 1.0799x over previous
import functools

import jax
import jax.numpy as jnp
from jax.experimental import pallas as pl
from jax.experimental.pallas import tpu as pltpu

_N = 4096
_TILE = 512


def _fused_body(attr_ref, mask_ref, w_ref, root_ref, bias_ref, out_ref, p_ref):
    i = pl.program_id(0)

    @pl.when(i == 0)
    def _():
        p_ref[...] = jnp.dot(attr_ref[...], w_ref[...],
                             preferred_element_type=jnp.float32)

    attr_tile = attr_ref[pl.ds(i * _TILE, _TILE), :]
    out_ref[...] = (
        jnp.dot(mask_ref[...], p_ref[...], preferred_element_type=jnp.float32)
        + jnp.dot(attr_tile, root_ref[...], preferred_element_type=jnp.float32)
        + bias_ref[...]
    )


@jax.jit
def kernel(node_attr, node_mask, node_weight, root, bias):
    n, d_in = node_attr.shape
    d_out = node_weight.shape[1]
    bias2d = bias.reshape(1, d_out)

    grid = (n // _TILE,)
    return pl.pallas_call(
        _fused_body,
        grid=grid,
        in_specs=[
            pl.BlockSpec((n, d_in), lambda i: (0, 0)),
            pl.BlockSpec((_TILE, n), lambda i: (i, 0)),
            pl.BlockSpec((d_in, d_out), lambda i: (0, 0)),
            pl.BlockSpec((d_in, d_out), lambda i: (0, 0)),
            pl.BlockSpec((1, d_out), lambda i: (0, 0)),
        ],
        out_specs=pl.BlockSpec((_TILE, d_out), lambda i: (i, 0)),
        out_shape=jax.ShapeDtypeStruct((n, d_out), jnp.float32),
        scratch_shapes=[pltpu.VMEM((n, d_out), jnp.float32)],
    )(node_attr, node_mask, node_weight, root, bias2d)
